# Initial kernel scaffold; baseline (speedup 1.0000x reference)
import jax
import jax.numpy as jnp
from jax.experimental import pallas as pl


def kernel(token_ids, token_embed, pos_embed):
    raise NotImplementedError("write your pallas kernel here")


# speedup vs baseline: 2.3709x; 2.3709x over previous
import jax
import jax.numpy as jnp
from jax import lax
from jax.experimental import pallas as pl
from jax.experimental.pallas import tpu as pltpu
from jax.experimental.pallas import tpu_sc as plsc

_INFO = plsc.get_sparse_core_info()
_NC = _INFO.num_cores
_NS = _INFO.num_subcores
_LANES = _INFO.num_lanes
_NW = _NC * _NS


def _make_body(T, D, seq_per_w):
    def _body(ids_hbm, tab_hbm, pos_hbm, out_hbm, idx_v, rows_v, pos_v, gsem):
        wid = lax.axis_index("s") * _NC + lax.axis_index("c")
        pltpu.sync_copy(pos_hbm, pos_v)

        def seq_body(s, carry):
            row0 = (wid * seq_per_w + s) * T
            pltpu.sync_copy(ids_hbm.at[pl.ds(row0, T)], idx_v)
            pltpu.async_copy(tab_hbm.at[idx_v], rows_v, gsem).wait()

            def add_body(t, c2):
                for j in range(D // _LANES):
                    sl = pl.ds(j * _LANES, _LANES)
                    rows_v[t, sl] = rows_v[t, sl] + pos_v[t, sl]
                return c2

            lax.fori_loop(0, T, add_body, 0)
            pltpu.sync_copy(rows_v, out_hbm.at[pl.ds(row0, T)])
            return carry

        lax.fori_loop(0, seq_per_w, seq_body, 0)

    return _body


def kernel(token_ids, token_embed, pos_embed):
    B, T = token_ids.shape
    V, D = token_embed.shape
    assert B % _NW == 0
    seq_per_w = B // _NW

    ids = token_ids.reshape(-1).astype(jnp.int32)

    out = pl.kernel(
        _make_body(T, D, seq_per_w),
        out_type=jax.ShapeDtypeStruct((B * T, D), jnp.float32),
        mesh=plsc.VectorSubcoreMesh(core_axis_name="c", subcore_axis_name="s"),
        scratch_types=[
            pltpu.VMEM((T,), jnp.int32),
            pltpu.VMEM((T, D), jnp.float32),
            pltpu.VMEM((T, D), jnp.float32),
            pltpu.SemaphoreType.DMA,
        ],
        compiler_params=pltpu.CompilerParams(use_tc_tiling_on_sc=False),
    )(ids, token_embed, pos_embed)
    return out.reshape(B, T, D)


# speedup vs baseline: 2.7371x; 1.1544x over previous
import jax
import jax.numpy as jnp
from jax import lax
from jax.experimental import pallas as pl
from jax.experimental.pallas import tpu as pltpu
from jax.experimental.pallas import tpu_sc as plsc

_INFO = plsc.get_sparse_core_info()
_NC = _INFO.num_cores
_NS = _INFO.num_subcores
_LANES = _INFO.num_lanes
_NW = _NC * _NS

_NBUF = 4


def _make_body(T, D, seq_per_w):
    n_chunks = seq_per_w

    def _body(ids_hbm, tab_hbm, pos_hbm, out_hbm, idx_v, rows_v, pos_v, gsem, wsem):
        wid = lax.axis_index("s") * _NC + lax.axis_index("c")
        base_row = wid * seq_per_w * T

        pltpu.sync_copy(pos_hbm, pos_v)

        def start_gather(c, slot):
            pltpu.sync_copy(ids_hbm.at[pl.ds(base_row + c * T, T)], idx_v.at[slot])
            pltpu.make_async_copy(
                tab_hbm.at[idx_v.at[slot]], rows_v.at[slot], gsem.at[slot]
            ).start()

        def wait_gather(slot):
            pltpu.make_async_copy(
                tab_hbm.at[idx_v.at[slot]], rows_v.at[slot], gsem.at[slot]
            ).wait()

        def start_write(c, slot):
            pltpu.make_async_copy(
                rows_v.at[slot], out_hbm.at[pl.ds(base_row + c * T, T)], wsem.at[slot]
            ).start()

        def wait_write(c, slot):
            pltpu.make_async_copy(
                rows_v.at[slot], out_hbm.at[pl.ds(base_row + c * T, T)], wsem.at[slot]
            ).wait()

        for b in range(2):
            start_gather(b, b)

        def group_body(g, carry):
            for b in range(_NBUF):
                c = g * _NBUF + b
                slot = b
                nslot = (b + 2) % _NBUF
                cp = c + 2

                @pl.when(cp < n_chunks)
                def _():
                    @pl.when(cp >= _NBUF)
                    def _():
                        wait_write(cp - _NBUF, nslot)

                    start_gather(cp, nslot)

                wait_gather(slot)

                def add_body(t, c2):
                    for j in range(D // _LANES):
                        sl = pl.ds(j * _LANES, _LANES)
                        rows_v[slot, t, sl] = rows_v[slot, t, sl] + pos_v[t, sl]
                    return c2

                lax.fori_loop(0, T, add_body, 0)
                start_write(c, slot)
            return carry

        lax.fori_loop(0, n_chunks // _NBUF, group_body, 0)

        for b in range(_NBUF):
            wait_write(n_chunks - _NBUF + b, b)

    return _body


def kernel(token_ids, token_embed, pos_embed):
    B, T = token_ids.shape
    V, D = token_embed.shape
    assert B % (_NW * _NBUF) == 0
    seq_per_w = B // _NW

    ids = token_ids.reshape(-1).astype(jnp.int32)

    out = pl.kernel(
        _make_body(T, D, seq_per_w),
        out_type=jax.ShapeDtypeStruct((B * T, D), jnp.float32),
        mesh=plsc.VectorSubcoreMesh(core_axis_name="c", subcore_axis_name="s"),
        scratch_types=[
            pltpu.VMEM((_NBUF, T), jnp.int32),
            pltpu.VMEM((_NBUF, T, D), jnp.float32),
            pltpu.VMEM((T, D), jnp.float32),
            pltpu.SemaphoreType.DMA((_NBUF,)),
            pltpu.SemaphoreType.DMA((_NBUF,)),
        ],
        compiler_params=pltpu.CompilerParams(use_tc_tiling_on_sc=False),
    )(ids, token_embed, pos_embed)
    return out.reshape(B, T, D)
